# baseline (speedup 1.0000x reference)
import functools

import jax
import jax.numpy as jnp
from jax import lax
from jax.experimental import pallas as pl
from jax.experimental.pallas import tpu as pltpu
from jax.experimental.pallas import tpu_sc as plsc

N_NODES = 10000
N_EDGES = 320000
HIDDEN = 128
NUM_RBF = 20

EBLK = 1280
CHUNK = 80
NSUB = 16
EDGES_PER_TILE = N_EDGES // NSUB
CHUNKS_PER_TILE = EDGES_PER_TILE // CHUNK
WB = 80
N_WB = N_NODES // WB
WB_ROUNDS = -(-N_WB // NSUB)


def _mlp_body(rbf_ref, ev_ref, w1_ref, b1_ref, w2_ref, b2_ref,
              o0, o1, o2, o3, o4):
    x = rbf_ref[...]
    h = lax.dot_general(x, w1_ref[...], (((1,), (1,)), ((), ())),
                        preferred_element_type=jnp.float32)
    h = h + b1_ref[...]
    h = h * lax.logistic(h)
    fw = lax.dot_general(h, w2_ref[...], (((1,), (1,)), ((), ())),
                         preferred_element_type=jnp.float32)
    fw = fw + b2_ref[...]
    fv2 = fw[:, 2 * HIDDEN:3 * HIDDEN]
    o0[...] = fw[:, 0:HIDDEN]
    o1[...] = fw[:, HIDDEN:2 * HIDDEN]
    o2[...] = fv2 * ev_ref[:, 0:1]
    o3[...] = fv2 * ev_ref[:, 1:2]
    o4[...] = fv2 * ev_ref[:, 2:3]


def _run_mlp(edge_rbf, edge_vector, W1, b1, W2, b2):
    nblk = N_EDGES // EBLK
    out_sds = jax.ShapeDtypeStruct((N_EDGES, HIDDEN), jnp.float32)
    eblock = pl.BlockSpec((EBLK, HIDDEN), lambda i: (i, 0))
    return pl.pallas_call(
        _mlp_body,
        grid=(nblk,),
        in_specs=[
            pl.BlockSpec((EBLK, NUM_RBF), lambda i: (i, 0)),
            pl.BlockSpec((EBLK, 3), lambda i: (i, 0)),
            pl.BlockSpec((HIDDEN, NUM_RBF), lambda i: (0, 0)),
            pl.BlockSpec((1, HIDDEN), lambda i: (0, 0)),
            pl.BlockSpec((3 * HIDDEN, HIDDEN), lambda i: (0, 0)),
            pl.BlockSpec((1, 3 * HIDDEN), lambda i: (0, 0)),
        ],
        out_specs=[eblock] * 5,
        out_shape=[out_sds] * 5,
    )(edge_rbf, edge_vector, W1, b1.reshape(1, HIDDEN),
      W2, b2.reshape(1, 3 * HIDDEN))


def _unit(sid, src_hbm, dst_hbm, feat_hbm, f1_hbm, f2_hbm, out_hbm,
          acc, src_v, dst_v, rows_v, f1_v, f2_v, stage_v, sem):
    is_vec = f2_hbm is not None

    for k in range(WB_ROUNDS):
        j = k * NSUB + sid

        @pl.when(j < N_WB)
        def _():
            r0 = j * WB
            pltpu.sync_copy(feat_hbm.at[pl.ds(r0, WB), :], stage_v)
            pltpu.sync_copy(stage_v, acc.at[pl.ds(r0, WB), :])
    plsc.subcore_barrier()

    def chunk_body(c, _):
        base = sid * EDGES_PER_TILE + c * CHUNK
        pltpu.sync_copy(src_hbm.at[pl.ds(base, CHUNK)], src_v)
        pltpu.sync_copy(dst_hbm.at[pl.ds(base, CHUNK)], dst_v)
        gather = pltpu.async_copy(feat_hbm.at[src_v], rows_v, sem)
        pltpu.sync_copy(f1_hbm.at[pl.ds(base, CHUNK), :], f1_v)
        if is_vec:
            pltpu.sync_copy(f2_hbm.at[pl.ds(base, CHUNK), :], f2_v)
        gather.wait()

        def edge_body(e, _):
            if is_vec:
                rows_v[e, :] = rows_v[e, :] * f1_v[e, :] + f2_v[e, :]
            else:
                rows_v[e, :] = rows_v[e, :] * f1_v[e, :]
            return 0

        lax.fori_loop(0, CHUNK, edge_body, 0)
        pltpu.sync_copy(rows_v, acc.at[dst_v], add=True)
        return 0

    lax.fori_loop(0, CHUNKS_PER_TILE, chunk_body, 0)
    plsc.subcore_barrier()

    for k in range(WB_ROUNDS):
        j = k * NSUB + sid

        @pl.when(j < N_WB)
        def _():
            r0 = j * WB
            pltpu.sync_copy(acc.at[pl.ds(r0, WB), :], stage_v)
            pltpu.sync_copy(stage_v, out_hbm.at[pl.ds(r0, WB), :])


def _make_sc_kernel():
    mesh = plsc.VectorSubcoreMesh(core_axis_name="c", subcore_axis_name="s")
    nf = jax.ShapeDtypeStruct((N_NODES, HIDDEN), jnp.float32)

    @functools.partial(
        pl.kernel,
        mesh=mesh,
        out_type=[nf, nf, nf, nf],
        scratch_types=[
            pltpu.VMEM_SHARED((N_NODES, HIDDEN), jnp.float32),
            pltpu.VMEM((CHUNK,), jnp.int32),
            pltpu.VMEM((CHUNK,), jnp.int32),
            pltpu.VMEM((CHUNK, HIDDEN), jnp.float32),
            pltpu.VMEM((CHUNK, HIDDEN), jnp.float32),
            pltpu.VMEM((CHUNK, HIDDEN), jnp.float32),
            pltpu.VMEM((WB, HIDDEN), jnp.float32),
            pltpu.SemaphoreType.DMA,
        ],
    )
    def sc_kernel(src, dst, fs, fv1, f20, f21, f22, sf, vf0, vf1, vf2,
                  so, vo0, vo1, vo2,
                  acc, src_v, dst_v, rows_v, f1_v, f2_v, stage_v, sem):
        core = lax.axis_index("c")
        sid = lax.axis_index("s")
        common = (acc, src_v, dst_v, rows_v, f1_v, f2_v, stage_v, sem)

        @pl.when(core == 0)
        def _():
            _unit(sid, src, dst, sf, fs, None, so, *common)
            _unit(sid, src, dst, vf1, fv1, f21, vo1, *common)

        @pl.when(core == 1)
        def _():
            _unit(sid, src, dst, vf0, fv1, f20, vo0, *common)
            _unit(sid, src, dst, vf2, fv1, f22, vo2, *common)

    return sc_kernel


_SC_KERNEL_CACHE = []


def kernel(scalar_features, vector_features, edge_index, edge_rbf,
           edge_vector, W1, b1, W2, b2):
    if not _SC_KERNEL_CACHE:
        _SC_KERNEL_CACHE.append(_make_sc_kernel())
    _SC_KERNEL = _SC_KERNEL_CACHE[0]
    fs, fv1, f20, f21, f22 = _run_mlp(edge_rbf, edge_vector, W1, b1, W2, b2)
    src = edge_index[0].astype(jnp.int32)
    dst = edge_index[1].astype(jnp.int32)
    vf_t = jnp.transpose(vector_features, (1, 0, 2))
    so, vo0, vo1, vo2 = _SC_KERNEL(
        src, dst, fs, fv1, f20, f21, f22,
        scalar_features, vf_t[0], vf_t[1], vf_t[2])
    vo = jnp.stack([vo0, vo1, vo2], axis=1)
    return (so, vo)


# speedup vs baseline: 1.8003x; 1.8003x over previous
import functools

import jax
import jax.numpy as jnp
from jax import lax
from jax.experimental import pallas as pl
from jax.experimental.pallas import tpu as pltpu
from jax.experimental.pallas import tpu_sc as plsc

N_NODES = 10000
N_EDGES = 320000
HIDDEN = 128
NUM_RBF = 20

EBLK = 1280
CHUNK = 40
NSUB = 16
EDGES_PER_TILE = N_EDGES // NSUB
CHUNKS_PER_TILE = EDGES_PER_TILE // CHUNK
GRP = 10
GRP_E = GRP * CHUNK
SUPER = 2 * GRP
NSUPER = CHUNKS_PER_TILE // SUPER
WB = 40
N_WB = N_NODES // WB
WB_ROUNDS = -(-N_WB // NSUB)


def _mlp_body(rbf_ref, ev_ref, w1_ref, b1_ref, w2_ref, b2_ref,
              o0, o1, o2, o3, o4):
    x = rbf_ref[...]
    h = lax.dot_general(x, w1_ref[...], (((1,), (1,)), ((), ())),
                        preferred_element_type=jnp.float32)
    h = h + b1_ref[...]
    h = h * lax.logistic(h)
    fw = lax.dot_general(h, w2_ref[...], (((1,), (1,)), ((), ())),
                         preferred_element_type=jnp.float32)
    fw = fw + b2_ref[...]
    fv2 = fw[:, 2 * HIDDEN:3 * HIDDEN]
    o0[...] = fw[:, 0:HIDDEN]
    o1[...] = fw[:, HIDDEN:2 * HIDDEN]
    o2[...] = fv2 * ev_ref[:, 0:1]
    o3[...] = fv2 * ev_ref[:, 1:2]
    o4[...] = fv2 * ev_ref[:, 2:3]


def _run_mlp(edge_rbf, edge_vector, W1, b1, W2, b2):
    nblk = N_EDGES // EBLK
    out_sds = jax.ShapeDtypeStruct((N_EDGES, HIDDEN), jnp.float32)
    eblock = pl.BlockSpec((EBLK, HIDDEN), lambda i: (i, 0))
    return pl.pallas_call(
        _mlp_body,
        grid=(nblk,),
        in_specs=[
            pl.BlockSpec((EBLK, NUM_RBF), lambda i: (i, 0)),
            pl.BlockSpec((EBLK, 3), lambda i: (i, 0)),
            pl.BlockSpec((HIDDEN, NUM_RBF), lambda i: (0, 0)),
            pl.BlockSpec((1, HIDDEN), lambda i: (0, 0)),
            pl.BlockSpec((3 * HIDDEN, HIDDEN), lambda i: (0, 0)),
            pl.BlockSpec((1, 3 * HIDDEN), lambda i: (0, 0)),
        ],
        out_specs=[eblock] * 5,
        out_shape=[out_sds] * 5,
    )(edge_rbf, edge_vector, W1, b1.reshape(1, HIDDEN),
      W2, b2.reshape(1, 3 * HIDDEN))


def _unit(sid, src_hbm, dst_hbm, feat_hbm, f1_hbm, f2_hbm, out_hbm,
          acc, src_g0, src_g1, dst_g0, dst_g1, rows_v, f1_v, f2_v,
          sem_ig, sem_flt, sem_g, sem_wb):
    is_vec = f2_hbm is not None
    tb = sid * EDGES_PER_TILE

    for r in range(WB_ROUNDS):
        j = r * NSUB + sid

        @pl.when(j < N_WB)
        def _():
            pltpu.async_copy(feat_hbm.at[pl.ds(j * WB, WB), :],
                             acc.at[pl.ds(j * WB, WB), :], sem_wb)
    for r in range(WB_ROUNDS):
        j = r * NSUB + sid

        @pl.when(j < N_WB)
        def _():
            pltpu.make_async_copy(feat_hbm.at[pl.ds(0, WB), :],
                                  acc.at[pl.ds(0, WB), :], sem_wb).wait()
    plsc.subcore_barrier()

    srcs = (src_g0, src_g1)
    dsts = (dst_g0, dst_g1)

    def idx_issue(g, buf):
        base = tb + g * GRP_E
        pltpu.async_copy(src_hbm.at[pl.ds(base, GRP_E)], srcs[buf],
                         sem_ig.at[buf])
        pltpu.async_copy(dst_hbm.at[pl.ds(base, GRP_E)], dsts[buf],
                         sem_ig.at[buf])

    def idx_drain(buf):
        pltpu.make_async_copy(src_hbm.at[pl.ds(0, GRP_E)], srcs[buf],
                              sem_ig.at[buf]).wait()
        pltpu.make_async_copy(dst_hbm.at[pl.ds(0, GRP_E)], dsts[buf],
                              sem_ig.at[buf]).wait()

    def flt_issue(cc, fb):
        base = tb + cc * CHUNK
        pltpu.async_copy(f1_hbm.at[pl.ds(base, CHUNK), :], f1_v.at[fb],
                         sem_flt.at[fb])
        if is_vec:
            pltpu.async_copy(f2_hbm.at[pl.ds(base, CHUNK), :], f2_v.at[fb],
                             sem_flt.at[fb])

    def flt_drain(fb):
        pltpu.make_async_copy(f1_hbm.at[pl.ds(0, CHUNK), :], f1_v.at[fb],
                              sem_flt.at[fb]).wait()
        if is_vec:
            pltpu.make_async_copy(f2_hbm.at[pl.ds(0, CHUNK), :],
                                  f2_v.at[fb], sem_flt.at[fb]).wait()

    def gat_issue(gbuf, lk, rb):
        pltpu.async_copy(
            feat_hbm.at[srcs[gbuf].at[pl.ds(lk * CHUNK, CHUNK)]],
            rows_v.at[rb], sem_g.at[rb])

    def gat_drain(rb):
        pltpu.make_async_copy(f1_hbm.at[pl.ds(0, CHUNK), :], rows_v.at[rb],
                              sem_g.at[rb]).wait()

    idx_issue(0, 0)
    idx_issue(1, 1)
    idx_drain(0)
    gat_issue(0, 0, 0)
    gat_issue(0, 1, 1)
    flt_issue(0, 0)
    flt_issue(1, 1)

    def super_body(s, _):
        not_last = s < NSUPER - 1
        for k in range(SUPER):
            rb = k % 4
            fb = k % 2
            if k == 8:
                idx_drain(1)
            if k == 18:
                @pl.when(not_last)
                def _():
                    idx_drain(0)
            if k <= 7:
                gat_issue(0, k + 2, (k + 2) % 4)
            elif k <= 17:
                gat_issue(1, k - 8, (k + 2) % 4)
            else:
                @pl.when(not_last)
                def _():
                    gat_issue(0, k - 18, (k + 2) % 4)
            flt_drain(fb)
            gat_drain(rb)
            rows = rows_v.at[rb]
            f1c = f1_v.at[fb]
            f2c = f2_v.at[fb]

            def row_body(e, _, rows=rows, f1c=f1c, f2c=f2c):
                if is_vec:
                    rows[e, :] = rows[e, :] * f1c[e, :] + f2c[e, :]
                else:
                    rows[e, :] = rows[e, :] * f1c[e, :]
                return 0

            lax.fori_loop(0, CHUNK, row_body, 0)
            if k <= 9:
                dsl = dst_g0.at[pl.ds(k * CHUNK, CHUNK)]
            else:
                dsl = dst_g1.at[pl.ds((k - 10) * CHUNK, CHUNK)]
            pltpu.sync_copy(rows, acc.at[dsl], add=True)
            cc = s * SUPER + k + 2
            if k <= 17:
                flt_issue(cc, fb)
            else:
                @pl.when(not_last)
                def _():
                    flt_issue(cc, fb)
            if k == 9:
                @pl.when(not_last)
                def _():
                    idx_issue(2 * s + 2, 0)
            if k == 19:
                @pl.when(not_last)
                def _():
                    idx_issue(2 * s + 3, 1)
        return 0

    lax.fori_loop(0, NSUPER, super_body, 0)
    plsc.subcore_barrier()

    for r in range(WB_ROUNDS):
        j = r * NSUB + sid

        @pl.when(j < N_WB)
        def _():
            pltpu.async_copy(acc.at[pl.ds(j * WB, WB), :],
                             out_hbm.at[pl.ds(j * WB, WB), :], sem_wb)
    for r in range(WB_ROUNDS):
        j = r * NSUB + sid

        @pl.when(j < N_WB)
        def _():
            pltpu.make_async_copy(acc.at[pl.ds(0, WB), :],
                                  out_hbm.at[pl.ds(0, WB), :], sem_wb).wait()


def _make_sc_kernel():
    mesh = plsc.VectorSubcoreMesh(core_axis_name="c", subcore_axis_name="s")
    nf = jax.ShapeDtypeStruct((N_NODES, HIDDEN), jnp.float32)

    @functools.partial(
        pl.kernel,
        mesh=mesh,
        out_type=[nf, nf, nf, nf],
        scratch_types=[
            pltpu.VMEM_SHARED((N_NODES, HIDDEN), jnp.float32),
            pltpu.VMEM((GRP_E,), jnp.int32),
            pltpu.VMEM((GRP_E,), jnp.int32),
            pltpu.VMEM((GRP_E,), jnp.int32),
            pltpu.VMEM((GRP_E,), jnp.int32),
            pltpu.VMEM((4, CHUNK, HIDDEN), jnp.float32),
            pltpu.VMEM((2, CHUNK, HIDDEN), jnp.float32),
            pltpu.VMEM((2, CHUNK, HIDDEN), jnp.float32),
            pltpu.SemaphoreType.DMA((2,)),
            pltpu.SemaphoreType.DMA((2,)),
            pltpu.SemaphoreType.DMA((4,)),
            pltpu.SemaphoreType.DMA,
        ],
    )
    def sc_kernel(src, dst, fs, fv1, f20, f21, f22, sf, vf0, vf1, vf2,
                  so, vo0, vo1, vo2,
                  acc, src_g0, src_g1, dst_g0, dst_g1, rows_v, f1_v, f2_v,
                  sem_ig, sem_flt, sem_g, sem_wb):
        core = lax.axis_index("c")
        sid = lax.axis_index("s")
        common = (acc, src_g0, src_g1, dst_g0, dst_g1, rows_v, f1_v, f2_v,
                  sem_ig, sem_flt, sem_g, sem_wb)

        @pl.when(core == 0)
        def _():
            _unit(sid, src, dst, sf, fs, None, so, *common)
            _unit(sid, src, dst, vf1, fv1, f21, vo1, *common)

        @pl.when(core == 1)
        def _():
            _unit(sid, src, dst, vf0, fv1, f20, vo0, *common)
            _unit(sid, src, dst, vf2, fv1, f22, vo2, *common)

    return sc_kernel


_SC_KERNEL_CACHE = []


def kernel(scalar_features, vector_features, edge_index, edge_rbf,
           edge_vector, W1, b1, W2, b2):
    if not _SC_KERNEL_CACHE:
        _SC_KERNEL_CACHE.append(_make_sc_kernel())
    _SC_KERNEL = _SC_KERNEL_CACHE[0]
    fs, fv1, f20, f21, f22 = _run_mlp(edge_rbf, edge_vector, W1, b1, W2, b2)
    src = edge_index[0].astype(jnp.int32)
    dst = edge_index[1].astype(jnp.int32)
    vf_t = jnp.transpose(vector_features, (1, 0, 2))
    so, vo0, vo1, vo2 = _SC_KERNEL(
        src, dst, fs, fv1, f20, f21, f22,
        scalar_features, vf_t[0], vf_t[1], vf_t[2])
    vo = jnp.stack([vo0, vo1, vo2], axis=1)
    return (so, vo)
